# baseline (speedup 1.0000x reference)
import functools

import jax
import jax.numpy as jnp
from jax import lax
from jax.experimental import pallas as pl
from jax.experimental.pallas import tpu as pltpu
from jax.experimental.pallas import tpu_sc as plsc

EMBED = 16
FIELD_OFFSET = 1_000_000
HALF_STRIDE = 16_000_000


def _sc_gather_t(x_t, table_flat, n_rows):
    info = plsc.get_sparse_core_info()
    nc, ns, lanes = info.num_cores, info.num_subcores, info.num_lanes
    nw = nc * ns
    mesh = plsc.VectorSubcoreMesh(core_axis_name="c", subcore_axis_name="s")

    @functools.partial(
        pl.kernel,
        mesh=mesh,
        out_type=jax.ShapeDtypeStruct((nw, n_rows), jnp.float32),
        scratch_types=[
            pltpu.VMEM((n_rows,), jnp.int32),
            pltpu.VMEM((n_rows,), jnp.int32),
            pltpu.VMEM((n_rows,), jnp.float32),
            pltpu.SemaphoreType.DMA,
        ],
        compiler_params=pltpu.CompilerParams(
            use_tc_tiling_on_sc=True, needs_layout_passes=False),
    )
    def gather_k(x_hbm, tab_hbm, out_hbm, xrow_v, idx_v, dst_v, sem):
        w = lax.axis_index("s") * nc + lax.axis_index("c")
        field = w // EMBED
        f = w % EMBED
        base = (f // 8) * HALF_STRIDE + (f % 8) * 128
        roff = field * FIELD_OFFSET

        chunk = 512
        n_chunks = n_rows // chunk
        groups_per_chunk = chunk // lanes

        def load_block(j):
            pltpu.sync_copy(
                x_hbm.at[field, pl.ds(j * chunk, chunk)],
                xrow_v.at[pl.ds(j * chunk, chunk)])

        def addr_block(j):
            def inner(t, _):
                for u in range(4):
                    sl = pl.ds(j * chunk + (t * 4 + u) * lanes, lanes)
                    r = xrow_v[sl] + roff
                    idx_v[sl] = (
                        lax.shift_left(lax.shift_right_logical(r, 7), 10)
                        + (r & 127) + base)
                return 0

            lax.fori_loop(0, groups_per_chunk // 4, inner, 0)

        load_block(0)
        addr_block(0)

        def fire_chunk(j, _):
            for k in range(chunk // 128):
                pltpu.async_copy(
                    tab_hbm.at[idx_v.at[pl.ds(j * chunk + k * 128, 128)]],
                    dst_v.at[pl.ds(j * chunk + k * 128, 128)],
                    sem,
                )

            @pl.when(j < n_chunks - 1)
            def _():
                load_block(j + 1)
                addr_block(j + 1)

            return 0

        lax.fori_loop(0, n_chunks, fire_chunk, 0)
        pltpu.make_async_copy(
            tab_hbm.at[pl.ds(0, n_rows)], dst_v, sem).wait()
        pltpu.sync_copy(dst_v, out_hbm.at[w])

    return gather_k(x_t, table_flat)


def _tc_mlp_t(h_t, W1t, b1, W2t, b2, W3t, b3, W4t, b4):
    n_rows = h_t.shape[1]
    blk = 16384
    grid = (n_rows // blk,)

    def mlp_k(h_ref, w1, c1, w2, c2, w3, c3, w4, c4, o_ref):
        a = h_ref[...]
        a = jnp.maximum(
            jnp.dot(w1[...], a, preferred_element_type=jnp.float32) + c1[...], 0.0)
        a = jnp.maximum(
            jnp.dot(w2[...], a, preferred_element_type=jnp.float32) + c2[...], 0.0)
        a = jnp.maximum(
            jnp.dot(w3[...], a, preferred_element_type=jnp.float32) + c3[...], 0.0)
        a = jnp.maximum(
            jnp.dot(w4[...], a, preferred_element_type=jnp.float32) + c4[...], 0.0)
        o_ref[...] = a

    full = lambda arr: pl.BlockSpec(arr.shape, lambda i: (0, 0))
    return pl.pallas_call(
        mlp_k,
        grid=grid,
        in_specs=[
            pl.BlockSpec((2 * EMBED, blk), lambda i: (0, i)),
            full(W1t), full(b1), full(W2t), full(b2),
            full(W3t), full(b3), full(W4t), full(b4),
        ],
        out_specs=pl.BlockSpec((1, blk), lambda i: (0, i)),
        out_shape=jax.ShapeDtypeStruct((1, n_rows), jnp.float32),
    )(h_t, W1t, b1, W2t, b2, W3t, b3, W4t, b4)


def kernel(x, table, W1, b1, W2, b2, W3, b3, W4, b4):
    n_rows = x.shape[0]
    x_t = x.T
    table_flat = (table.T.reshape(2, 8, 15625, 128)
                  .transpose(0, 2, 1, 3).reshape(-1))
    h_t = _sc_gather_t(x_t, table_flat, n_rows)
    out_t = _tc_mlp_t(
        h_t,
        W1.T, b1.reshape(-1, 1),
        W2.T, b2.reshape(-1, 1),
        W3.T, b3.reshape(-1, 1),
        W4.T, b4.reshape(-1, 1),
    )
    return out_t.reshape(n_rows, 1)


# speedup vs baseline: 1.1517x; 1.1517x over previous
import functools

import jax
import jax.numpy as jnp
from jax import lax
from jax.experimental import pallas as pl
from jax.experimental.pallas import tpu as pltpu
from jax.experimental.pallas import tpu_sc as plsc

EMBED = 16
FIELD_OFFSET = 1_000_000
HALF_STRIDE = 16_000_000


def _sc_gather_t(x_t, table_flat, n_rows):
    info = plsc.get_sparse_core_info()
    nc, ns, lanes = info.num_cores, info.num_subcores, info.num_lanes
    nw = nc * ns
    mesh = plsc.VectorSubcoreMesh(core_axis_name="c", subcore_axis_name="s")

    @functools.partial(
        pl.kernel,
        mesh=mesh,
        out_type=jax.ShapeDtypeStruct((nw, n_rows), jnp.float32),
        scratch_types=[
            pltpu.VMEM((n_rows,), jnp.int32),
            pltpu.VMEM((n_rows,), jnp.int32),
            pltpu.VMEM((n_rows,), jnp.float32),
            pltpu.SemaphoreType.DMA,
        ],
        compiler_params=pltpu.CompilerParams(
            use_tc_tiling_on_sc=True, needs_layout_passes=False),
    )
    def gather_k(x_hbm, tab_hbm, out_hbm, xrow_v, idx_v, dst_v, sem):
        w = lax.axis_index("s") * nc + lax.axis_index("c")
        field = w // EMBED
        f = w % EMBED
        pltpu.sync_copy(x_hbm.at[field], xrow_v)
        base = (f // 8) * HALF_STRIDE + (f % 8) * 128
        roff = field * FIELD_OFFSET

        chunk = 512
        n_chunks = n_rows // chunk
        groups_per_chunk = chunk // lanes

        def addr_block(j):
            def inner(t, _):
                for u in range(4):
                    sl = pl.ds(j * chunk + (t * 4 + u) * lanes, lanes)
                    r = xrow_v[sl] + roff
                    idx_v[sl] = (
                        lax.shift_left(lax.shift_right_logical(r, 7), 10)
                        + (r & 127) + base)
                return 0

            lax.fori_loop(0, groups_per_chunk // 4, inner, 0)

        addr_block(0)

        def fire_chunk(j, _):
            for k in range(chunk // 128):
                pltpu.async_copy(
                    tab_hbm.at[idx_v.at[pl.ds(j * chunk + k * 128, 128)]],
                    dst_v.at[pl.ds(j * chunk + k * 128, 128)],
                    sem,
                )

            @pl.when(j < n_chunks - 1)
            def _():
                addr_block(j + 1)

            return 0

        lax.fori_loop(0, n_chunks, fire_chunk, 0)
        pltpu.make_async_copy(
            tab_hbm.at[pl.ds(0, n_rows)], dst_v, sem).wait()
        pltpu.sync_copy(dst_v, out_hbm.at[w])

    return gather_k(x_t, table_flat)


def _tc_mlp_t(h_t, W1t, b1, W2t, b2, W3t, b3, W4t, b4):
    n_rows = h_t.shape[1]
    blk = 16384
    grid = (n_rows // blk,)

    def mlp_k(h_ref, w1, c1, w2, c2, w3, c3, w4, c4, o_ref):
        a = h_ref[...]
        a = jnp.maximum(
            jnp.dot(w1[...], a, preferred_element_type=jnp.float32) + c1[...], 0.0)
        a = jnp.maximum(
            jnp.dot(w2[...], a, preferred_element_type=jnp.float32) + c2[...], 0.0)
        a = jnp.maximum(
            jnp.dot(w3[...], a, preferred_element_type=jnp.float32) + c3[...], 0.0)
        a = jnp.maximum(
            jnp.dot(w4[...], a, preferred_element_type=jnp.float32) + c4[...], 0.0)
        o_ref[...] = a

    full = lambda arr: pl.BlockSpec(arr.shape, lambda i: (0, 0))
    return pl.pallas_call(
        mlp_k,
        grid=grid,
        in_specs=[
            pl.BlockSpec((2 * EMBED, blk), lambda i: (0, i)),
            full(W1t), full(b1), full(W2t), full(b2),
            full(W3t), full(b3), full(W4t), full(b4),
        ],
        out_specs=pl.BlockSpec((1, blk), lambda i: (0, i)),
        out_shape=jax.ShapeDtypeStruct((1, n_rows), jnp.float32),
    )(h_t, W1t, b1, W2t, b2, W3t, b3, W4t, b4)


def kernel(x, table, W1, b1, W2, b2, W3, b3, W4, b4):
    n_rows = x.shape[0]
    x_t = x.T
    table_flat = (table.T.reshape(2, 8, 15625, 128)
                  .transpose(0, 2, 1, 3).reshape(-1))
    h_t = _sc_gather_t(x_t, table_flat, n_rows)
    out_t = _tc_mlp_t(
        h_t,
        W1.T, b1.reshape(-1, 1),
        W2.T, b2.reshape(-1, 1),
        W3.T, b3.reshape(-1, 1),
        W4.T, b4.reshape(-1, 1),
    )
    return out_t.reshape(n_rows, 1)
